# baseline (speedup 1.0000x reference)
import jax
import jax.numpy as jnp
from jax.experimental import pallas as pl
from jax.experimental.pallas import tpu as pltpu

_N_CHUNKS = 8


def _dma_body(x_ref, o_ref, *sems):
    rows = x_ref.shape[0]
    chunk = rows // _N_CHUNKS
    copies = [
        pltpu.make_async_copy(
            x_ref.at[pl.ds(i * chunk, chunk), :],
            o_ref.at[pl.ds(i * chunk, chunk), :],
            sems[i],
        )
        for i in range(_N_CHUNKS)
    ]
    for c in copies:
        c.start()
    for c in copies:
        c.wait()


def kernel(x):
    return pl.pallas_call(
        _dma_body,
        in_specs=[pl.BlockSpec(memory_space=pl.ANY)],
        out_specs=pl.BlockSpec(memory_space=pl.ANY),
        out_shape=jax.ShapeDtypeStruct(x.shape, x.dtype),
        scratch_shapes=[pltpu.SemaphoreType.DMA] * _N_CHUNKS,
    )(x)


# speedup vs baseline: 47.1995x; 47.1995x over previous
import jax
import jax.numpy as jnp
from jax.experimental import pallas as pl

_BLOCK_ROWS = 1024


def _copy_body(x_ref, o_ref):
    o_ref[...] = x_ref[...]


def kernel(x):
    rows, cols = x.shape
    grid = (rows // _BLOCK_ROWS,)
    return pl.pallas_call(
        _copy_body,
        grid=grid,
        in_specs=[pl.BlockSpec((_BLOCK_ROWS, cols), lambda i: (i, 0))],
        out_specs=pl.BlockSpec((_BLOCK_ROWS, cols), lambda i: (i, 0)),
        out_shape=jax.ShapeDtypeStruct((rows, cols), x.dtype),
    )(x)
